# baseline (speedup 1.0000x reference)
import functools
import math

import jax
import jax.numpy as jnp
from jax.experimental import pallas as pl
from jax.experimental.pallas import tpu as pltpu


def _make_body(B, BLK, nsteps):
    def body(x_ref, w1_ref, b1_ref, w2_ref, b2_ref, out_t_ref, out_p_ref,
             yt_buf, yp_buf, sem_t, sem_p):
        i = pl.program_id(0)
        s = jax.lax.rem(i, 2)
        CB = x_ref.shape[0] * x_ref.shape[1]
        D = yt_buf.shape[-1]

        def copy(buf, hbm, sem, slot, b, step):
            return pltpu.make_async_copy(
                buf.at[slot, b],
                hbm.at[b, pl.ds(step * BLK, BLK), :],
                sem.at[slot, b],
            )

        @pl.when(i >= 2)
        def _():
            for b in range(B):
                copy(yt_buf, out_t_ref, sem_t, s, b, i - 2).wait()
                copy(yp_buf, out_p_ref, sem_p, s, b, i - 2).wait()

        xb = x_ref[:, :, pl.ds(i * BLK, BLK)].reshape(CB, BLK)
        h = jax.lax.dot_general(xb, w1_ref[...], (((0,), (0,)), ((), ())),
                                preferred_element_type=jnp.float32)
        h = h + b1_ref[...]
        g = (h * (1.0 + jax.lax.erf(h * (1.0 / math.sqrt(2.0))))
             ).astype(jnp.bfloat16)
        w2 = w2_ref[...]
        b2 = b2_ref[...]
        for b in range(B):
            y = jnp.dot(g[:, b * D:(b + 1) * D], w2,
                        preferred_element_type=jnp.float32) + b2
            yt_buf[s, b] = y[:, :D]
            copy(yt_buf, out_t_ref, sem_t, s, b, i).start()
            yp_buf[s, b] = y[:, D:]
            copy(yp_buf, out_p_ref, sem_p, s, b, i).start()

        @pl.when(i == nsteps - 1)
        def _():
            for b in range(B):
                copy(yt_buf, out_t_ref, sem_t, 1 - s, b, i - 1).wait()
                copy(yp_buf, out_p_ref, sem_p, 1 - s, b, i - 1).wait()
                copy(yt_buf, out_t_ref, sem_t, s, b, i).wait()
                copy(yp_buf, out_p_ref, sem_p, s, b, i).wait()
    return body


@functools.partial(jax.jit, static_argnames=())
def kernel(points, W1t, b1t, W2t, b2t, W1p, b1p, W2p, b2p):
    B, N, C = points.shape
    D = W2t.shape[0]
    H = W1t.shape[0]

    xT = jnp.transpose(points, (2, 0, 1))

    w1c = jnp.concatenate([W1t.T, W1p.T], axis=1)
    eye_b = jnp.eye(B, dtype=jnp.float32)
    w1big = (w1c[:, None, None, :] * eye_b[None, :, :, None]
             ).reshape(C * B, B * 2 * H)
    b1c = jnp.concatenate([b1t, b1p])
    b1big = jnp.tile(b1c, B)[None, :]

    w2c = jnp.zeros((2 * H, 2 * D), jnp.float32)
    w2c = w2c.at[:H, :D].set(0.5 * W2t.T).at[H:, D:].set(0.5 * W2p.T)
    w2c = w2c.astype(jnp.bfloat16)
    b2c = jnp.concatenate([b2t, b2p])[None, :]

    BLK = 512
    nsteps = N // BLK
    grid = (nsteps,)

    out_t, out_p = pl.pallas_call(
        _make_body(B, BLK, nsteps),
        grid=grid,
        in_specs=[
            pl.BlockSpec((C, B, N), lambda i: (0, 0, 0)),
            pl.BlockSpec((C * B, B * 2 * H), lambda i: (0, 0)),
            pl.BlockSpec((1, B * 2 * H), lambda i: (0, 0)),
            pl.BlockSpec((2 * H, 2 * D), lambda i: (0, 0)),
            pl.BlockSpec((1, 2 * D), lambda i: (0, 0)),
        ],
        out_specs=[
            pl.BlockSpec(memory_space=pltpu.MemorySpace.HBM),
            pl.BlockSpec(memory_space=pltpu.MemorySpace.HBM),
        ],
        out_shape=[
            jax.ShapeDtypeStruct((B, N, D), jnp.float32),
            jax.ShapeDtypeStruct((B, N, D), jnp.float32),
        ],
        scratch_shapes=[
            pltpu.VMEM((2, B, BLK, D), jnp.float32),
            pltpu.VMEM((2, B, BLK, D), jnp.float32),
            pltpu.SemaphoreType.DMA((2, B)),
            pltpu.SemaphoreType.DMA((2, B)),
        ],
    )(xT, w1big, b1big, w2c, b2c)

    return (out_t, out_p)


# speedup vs baseline: 1.0142x; 1.0142x over previous
import functools
import math

import jax
import jax.numpy as jnp
from jax.experimental import pallas as pl
from jax.experimental.pallas import tpu as pltpu


def _make_body(B, BLK, nsteps):
    def body(x_ref, w1t_ref, b1t_ref, w2t_ref, b2t_ref,
             w1p_ref, b1p_ref, w2p_ref, b2p_ref,
             out_t_ref, out_p_ref,
             w1_s, b1_s, w2_s, b2_s, yt_buf, yp_buf, sem_t, sem_p):
        i = pl.program_id(0)
        s = jax.lax.rem(i, 2)
        C = x_ref.shape[0]
        CB = C * B
        D = yt_buf.shape[-1]
        H = w2t_ref.shape[1]

        @pl.when(i == 0)
        def _():
            w1c = jnp.concatenate(
                [w1t_ref[...].T, w1p_ref[...].T], axis=1)
            w1_s[...] = jnp.zeros_like(w1_s)
            w2_s[...] = jnp.zeros_like(w2_s)
            for c in range(C):
                for b in range(B):
                    w1_s[c * B + b:c * B + b + 1,
                         b * 2 * H:(b + 1) * 2 * H] = w1c[c:c + 1, :]
            w2_s[:H, :D] = 0.5 * w2t_ref[...].T
            w2_s[H:, D:] = 0.5 * w2p_ref[...].T
            b1c = jnp.concatenate([b1t_ref[...], b1p_ref[...]], axis=1)
            for b in range(B):
                b1_s[:, b * 2 * H:(b + 1) * 2 * H] = b1c
            b2_s[:, :D] = b2t_ref[...]
            b2_s[:, D:] = b2p_ref[...]

        def copy(buf, hbm, sem, slot, b, step):
            return pltpu.make_async_copy(
                buf.at[slot, b],
                hbm.at[b, pl.ds(step * BLK, BLK), :],
                sem.at[slot, b],
            )

        @pl.when(i >= 2)
        def _():
            for b in range(B):
                copy(yt_buf, out_t_ref, sem_t, s, b, i - 2).wait()
                copy(yp_buf, out_p_ref, sem_p, s, b, i - 2).wait()

        xb = x_ref[:, :, pl.ds(i * BLK, BLK)].reshape(CB, BLK)
        h = jax.lax.dot_general(xb, w1_s[...], (((0,), (0,)), ((), ())),
                                preferred_element_type=jnp.float32)
        h = h + b1_s[...]
        g = h * (1.0 + jax.lax.erf(h * (1.0 / math.sqrt(2.0))))
        w2 = w2_s[...]
        b2 = b2_s[...]
        for b in range(B):
            y = jnp.dot(g[:, b * D:(b + 1) * D], w2,
                        preferred_element_type=jnp.float32) + b2
            yt_buf[s, b] = y[:, :D]
            copy(yt_buf, out_t_ref, sem_t, s, b, i).start()
            yp_buf[s, b] = y[:, D:]
            copy(yp_buf, out_p_ref, sem_p, s, b, i).start()

        @pl.when(i == nsteps - 1)
        def _():
            for b in range(B):
                copy(yt_buf, out_t_ref, sem_t, 1 - s, b, i - 1).wait()
                copy(yp_buf, out_p_ref, sem_p, 1 - s, b, i - 1).wait()
                copy(yt_buf, out_t_ref, sem_t, s, b, i).wait()
                copy(yp_buf, out_p_ref, sem_p, s, b, i).wait()
    return body


@functools.partial(jax.jit, static_argnames=())
def kernel(points, W1t, b1t, W2t, b2t, W1p, b1p, W2p, b2p):
    B, N, C = points.shape
    D = W2t.shape[0]
    H = W1t.shape[0]

    xT = jnp.transpose(points, (2, 0, 1))

    BLK = 512
    nsteps = N // BLK
    grid = (nsteps,)

    def full(shape):
        return pl.BlockSpec(shape, lambda i: tuple(0 for _ in shape))

    out_t, out_p = pl.pallas_call(
        _make_body(B, BLK, nsteps),
        grid=grid,
        in_specs=[
            full((C, B, N)),
            full((H, C)), full((1, H)), full((D, H)), full((1, D)),
            full((H, C)), full((1, H)), full((D, H)), full((1, D)),
        ],
        out_specs=[
            pl.BlockSpec(memory_space=pltpu.MemorySpace.HBM),
            pl.BlockSpec(memory_space=pltpu.MemorySpace.HBM),
        ],
        out_shape=[
            jax.ShapeDtypeStruct((B, N, D), jnp.float32),
            jax.ShapeDtypeStruct((B, N, D), jnp.float32),
        ],
        scratch_shapes=[
            pltpu.VMEM((C * B, B * 2 * H), jnp.float32),
            pltpu.VMEM((1, B * 2 * H), jnp.float32),
            pltpu.VMEM((2 * H, 2 * D), jnp.float32),
            pltpu.VMEM((1, 2 * D), jnp.float32),
            pltpu.VMEM((2, B, BLK, D), jnp.float32),
            pltpu.VMEM((2, B, BLK, D), jnp.float32),
            pltpu.SemaphoreType.DMA((2, B)),
            pltpu.SemaphoreType.DMA((2, B)),
        ],
    )(xT, W1t, b1t[None, :], W2t, b2t[None, :],
      W1p, b1p[None, :], W2p, b2p[None, :])

    return (out_t, out_p)
